# baseline (speedup 1.0000x reference)
import functools

import jax
import jax.numpy as jnp
from jax import lax
from jax.experimental import pallas as pl
from jax.experimental.pallas import tpu as pltpu

N = 100000
B = 512
F = 128
C = 5000
STEPS = N // C
W = 64

_HI = lax.Precision.HIGHEST


def _body(nf_ref, ids_row_ref, g_ref, wl_ref, bl_ref,
          wp_ref, bp_ref, wih_ref, whh_ref, bih_ref, bhh_ref,
          out_ref, s_ref, m_ref, d_ref, v_ref):
    i = pl.program_id(0)

    @pl.when(i == 0)
    def _init():
        g_relu = jnp.maximum(g_ref[...], 0.0)
        w1 = wl_ref[0:1, :]
        s_ref[...] = jnp.sum(g_relu * w1, axis=1, keepdims=True)
        m_ref[...] = jnp.full((1, 1), -1e30, jnp.float32)
        d_ref[...] = jnp.zeros((B, 1), jnp.float32)
        v_ref[...] = jnp.zeros((B, F), jnp.float32)

    chunk = nf_ref[...]
    seg_row = ids_row_ref[0]

    lo = jnp.min(seg_row)
    hi = jnp.max(seg_row)
    lo8 = jnp.minimum((lo // 8) * 8, B - W)
    narrow = (hi - lo8) < W

    w2 = wl_ref[1:2, :]
    t = jax.lax.dot_general(w2, chunk, (((1,), (1,)), ((), ())),
                            preferred_element_type=jnp.float32)

    def _accumulate(eq, s_win):
        s_g = jnp.sum(jnp.where(eq, s_win, 0.0), axis=0, keepdims=True)
        z = t + s_g + bl_ref[...]
        z = jnp.where(z >= 0.0, z, 0.01 * z)
        m_old = m_ref[...]
        m_new = jnp.maximum(m_old, jnp.max(z, axis=(0, 1), keepdims=True))
        fac = jnp.exp(m_old - m_new)
        m_ref[...] = m_new
        ez = jnp.exp(z - m_new)
        oh_scaled = jnp.where(eq, ez, 0.0)
        d_part = jnp.sum(oh_scaled, axis=1, keepdims=True)
        v_part = jax.lax.dot_general(
            oh_scaled, chunk, (((1,), (0,)), ((), ())),
            preferred_element_type=jnp.float32)

        @pl.when(m_new[0, 0] > m_old[0, 0])
        def _rescale():
            d_ref[...] = d_ref[...] * fac
            v_ref[...] = v_ref[...] * fac
        return d_part, v_part

    @pl.when(narrow)
    def _narrow():
        rel = seg_row - lo8
        eq = lax.broadcasted_iota(jnp.int32, (W, C), 0) == rel
        d_part, v_part = _accumulate(eq, s_ref[pl.ds(lo8, W), :])
        d_ref[pl.ds(lo8, W), :] += d_part
        v_ref[pl.ds(lo8, W), :] += v_part

    @pl.when(jnp.logical_not(narrow))
    def _full():
        eq = lax.broadcasted_iota(jnp.int32, (B, C), 0) == seg_row
        d_part, v_part = _accumulate(eq, s_ref[...])
        d_ref[...] += d_part
        v_ref[...] += v_part

    @pl.when(i == STEPS - 1)
    def _finish():
        d = d_ref[...]
        nonempty = (d > 0.0).astype(jnp.float32)
        d_safe = jnp.where(d > 0.0, d, 1.0)
        wmean = v_ref[...] / d_safe
        g_repr = jax.lax.dot_general(
            wmean, wp_ref[...], (((1,), (1,)), ((), ())),
            preferred_element_type=jnp.float32, precision=_HI)
        g_repr = g_repr + nonempty * bp_ref[...]
        context = jnp.where(g_repr > 0.0, g_repr, jnp.exp(g_repr) - 1.0)
        g_prev = g_ref[...]
        gi = jax.lax.dot_general(
            context, wih_ref[...], (((1,), (1,)), ((), ())),
            preferred_element_type=jnp.float32, precision=_HI) + bih_ref[...]
        gh = jax.lax.dot_general(
            g_prev, whh_ref[...], (((1,), (1,)), ((), ())),
            preferred_element_type=jnp.float32, precision=_HI) + bhh_ref[...]
        r = jax.nn.sigmoid(gi[:, 0:F] + gh[:, 0:F])
        zg = jax.nn.sigmoid(gi[:, F:2 * F] + gh[:, F:2 * F])
        n = jnp.tanh(gi[:, 2 * F:3 * F] + r * gh[:, 2 * F:3 * F])
        out_ref[...] = (1.0 - zg) * n + zg * g_prev


@jax.jit
def kernel(node_feats, g_feats, segment_ids, W_logit, b_logit,
           W_proj, b_proj, W_ih, W_hh, b_ih, b_hh):
    ids = segment_ids.astype(jnp.int32)
    ids_row = ids.reshape(STEPS, 1, C)
    wl = W_logit.reshape(2, F)
    bl = b_logit.reshape(1, 1)
    bp = b_proj.reshape(1, F)
    bih = b_ih.reshape(1, 3 * F)
    bhh = b_hh.reshape(1, 3 * F)

    const = lambda shape: pl.BlockSpec(shape, lambda i: (0,) * len(shape))
    return pl.pallas_call(
        _body,
        grid=(STEPS,),
        in_specs=[
            pl.BlockSpec((C, F), lambda i: (i, 0)),
            pl.BlockSpec((1, 1, C), lambda i: (i, 0, 0)),
            const((B, F)),
            const((2, F)),
            const((1, 1)),
            const((F, F)),
            const((1, F)),
            const((3 * F, F)),
            const((3 * F, F)),
            const((1, 3 * F)),
            const((1, 3 * F)),
        ],
        out_specs=const((B, F)),
        out_shape=jax.ShapeDtypeStruct((B, F), jnp.float32),
        scratch_shapes=[
            pltpu.VMEM((B, 1), jnp.float32),
            pltpu.VMEM((1, 1), jnp.float32),
            pltpu.VMEM((B, 1), jnp.float32),
            pltpu.VMEM((B, F), jnp.float32),
        ],
        compiler_params=pltpu.CompilerParams(
            dimension_semantics=("arbitrary",)),
    )(node_feats, ids_row, g_feats, wl, bl,
      W_proj, bp, W_ih, W_hh, bih, bhh)


# speedup vs baseline: 1.0060x; 1.0060x over previous
import functools

import jax
import jax.numpy as jnp
from jax import lax
from jax.experimental import pallas as pl
from jax.experimental.pallas import tpu as pltpu

N = 100000
B = 512
F = 128
C = 5000
STEPS = N // C
W = 64

_HI = lax.Precision.HIGHEST


def _body(nf_ref, ids_row_ref, g_ref, wl_ref, bl_ref,
          wp_ref, bp_ref, wih_ref, whh_ref, bih_ref, bhh_ref,
          out_ref, s_ref, m_ref, d_ref, v_ref):
    i = pl.program_id(0)

    @pl.when(i == 0)
    def _init():
        g_relu = jnp.maximum(g_ref[...], 0.0)
        w1 = wl_ref[0:1, :]
        s_ref[...] = jnp.sum(g_relu * w1, axis=1, keepdims=True)
        m_ref[...] = jnp.full((1, 1), -1e30, jnp.float32)
        d_ref[...] = jnp.zeros((B, 1), jnp.float32)
        v_ref[...] = jnp.zeros((B, F), jnp.float32)

    chunk = nf_ref[...]
    seg_row = ids_row_ref[0]

    lo = jnp.min(seg_row)
    hi = jnp.max(seg_row)
    lo8 = jnp.minimum((lo // 8) * 8, B - W)
    narrow = (hi - lo8) < W

    w2 = wl_ref[1:2, :]
    t = jax.lax.dot_general(w2, chunk, (((1,), (1,)), ((), ())),
                            preferred_element_type=jnp.float32)

    def _accumulate(eq, s_win):
        s_g = jnp.sum(jnp.where(eq, s_win, 0.0), axis=0, keepdims=True)
        z = t + s_g + bl_ref[...]
        z = jnp.where(z >= 0.0, z, 0.01 * z)
        m_old = m_ref[...]
        m_new = jnp.maximum(m_old, jnp.max(z, axis=(0, 1), keepdims=True))
        fac = jnp.exp(m_old - m_new)
        m_ref[...] = m_new
        ez = jnp.exp(z - m_new)
        oh_scaled = jnp.where(eq, ez, 0.0)
        d_part = jnp.sum(oh_scaled, axis=1, keepdims=True)
        v_part = jax.lax.dot_general(
            oh_scaled, chunk, (((1,), (0,)), ((), ())),
            preferred_element_type=jnp.float32)

        d_ref[...] = d_ref[...] * fac
        v_ref[...] = v_ref[...] * fac
        return d_part, v_part

    @pl.when(narrow)
    def _narrow():
        rel = seg_row - lo8
        eq = lax.broadcasted_iota(jnp.int32, (W, C), 0) == rel
        d_part, v_part = _accumulate(eq, s_ref[pl.ds(lo8, W), :])
        d_ref[pl.ds(lo8, W), :] += d_part
        v_ref[pl.ds(lo8, W), :] += v_part

    @pl.when(jnp.logical_not(narrow))
    def _full():
        eq = lax.broadcasted_iota(jnp.int32, (B, C), 0) == seg_row
        d_part, v_part = _accumulate(eq, s_ref[...])
        d_ref[...] += d_part
        v_ref[...] += v_part

    @pl.when(i == STEPS - 1)
    def _finish():
        d = d_ref[...]
        nonempty = (d > 0.0).astype(jnp.float32)
        d_safe = jnp.where(d > 0.0, d, 1.0)
        wmean = v_ref[...] / d_safe
        g_repr = jax.lax.dot_general(
            wmean, wp_ref[...], (((1,), (1,)), ((), ())),
            preferred_element_type=jnp.float32, precision=_HI)
        g_repr = g_repr + nonempty * bp_ref[...]
        context = jnp.where(g_repr > 0.0, g_repr, jnp.exp(g_repr) - 1.0)
        g_prev = g_ref[...]
        gi = jax.lax.dot_general(
            context, wih_ref[...], (((1,), (1,)), ((), ())),
            preferred_element_type=jnp.float32, precision=_HI) + bih_ref[...]
        gh = jax.lax.dot_general(
            g_prev, whh_ref[...], (((1,), (1,)), ((), ())),
            preferred_element_type=jnp.float32, precision=_HI) + bhh_ref[...]
        r = jax.nn.sigmoid(gi[:, 0:F] + gh[:, 0:F])
        zg = jax.nn.sigmoid(gi[:, F:2 * F] + gh[:, F:2 * F])
        n = jnp.tanh(gi[:, 2 * F:3 * F] + r * gh[:, 2 * F:3 * F])
        out_ref[...] = (1.0 - zg) * n + zg * g_prev


@jax.jit
def kernel(node_feats, g_feats, segment_ids, W_logit, b_logit,
           W_proj, b_proj, W_ih, W_hh, b_ih, b_hh):
    ids = segment_ids.astype(jnp.int32)
    ids_row = ids.reshape(STEPS, 1, C)
    wl = W_logit.reshape(2, F)
    bl = b_logit.reshape(1, 1)
    bp = b_proj.reshape(1, F)
    bih = b_ih.reshape(1, 3 * F)
    bhh = b_hh.reshape(1, 3 * F)

    const = lambda shape: pl.BlockSpec(shape, lambda i: (0,) * len(shape))
    return pl.pallas_call(
        _body,
        grid=(STEPS,),
        in_specs=[
            pl.BlockSpec((C, F), lambda i: (i, 0)),
            pl.BlockSpec((1, 1, C), lambda i: (i, 0, 0)),
            const((B, F)),
            const((2, F)),
            const((1, 1)),
            const((F, F)),
            const((1, F)),
            const((3 * F, F)),
            const((3 * F, F)),
            const((1, 3 * F)),
            const((1, 3 * F)),
        ],
        out_specs=const((B, F)),
        out_shape=jax.ShapeDtypeStruct((B, F), jnp.float32),
        scratch_shapes=[
            pltpu.VMEM((B, 1), jnp.float32),
            pltpu.VMEM((1, 1), jnp.float32),
            pltpu.VMEM((B, 1), jnp.float32),
            pltpu.VMEM((B, F), jnp.float32),
        ],
        compiler_params=pltpu.CompilerParams(
            dimension_semantics=("arbitrary",)),
    )(node_feats, ids_row, g_feats, wl, bl,
      W_proj, bp, W_ih, W_hh, bih, bhh)
